# baseline (speedup 1.0000x reference)
import math

import jax
import jax.numpy as jnp
from jax import lax
from jax.experimental import pallas as pl
from jax.experimental.pallas import tpu as pltpu
from jax.experimental.pallas import tpu_sc as plsc

D_MODEL = 128
SEQ = 50
SCALE = math.sqrt(128.0)

NC, NS = 2, 16
NW = NC * NS
C = 128
NCHUNK = SEQ
NB = 5
LEAD = 4
BATCH = NW * C


def _scale_chunk(buf):
    def row(i, carry):
        for h in range(D_MODEL // 16):
            sl = (i, pl.ds(h * 16, 16))
            buf[sl] = buf[sl] * SCALE
        return carry
    lax.fori_loop(0, C, row, 0, unroll=2)


def _sc_body(x_hbm, table_hbm, out_hbm, idx_v, rows, *sems):
    gs = sems[:2 * NB]
    ss = sems[2 * NB:]
    wid = lax.axis_index("s") * NC + lax.axis_index("c")
    base = wid * C

    pltpu.sync_copy(x_hbm.at[wid], idx_v)

    class _Pair:
        def __init__(self, copies):
            self.copies = copies

        def start(self):
            for c in self.copies:
                c.start()

        def wait(self):
            for c in self.copies:
                c.wait()

    def gather(j, b):
        return _Pair([
            pltpu.make_async_copy(
                table_hbm.at[idx_v.at[j, h]],
                rows.at[b, pl.ds(h * (C // 2), C // 2)],
                gs[2 * b + h])
            for h in range(2)
        ])

    def scatter(j, b):
        return pltpu.make_async_copy(
            rows.at[b], out_hbm.at[j, pl.ds(base, C)], ss[b])

    for b in range(NB):
        gather(b, b).start()

    def outer(g, carry):
        for b in range(NB):
            j = g * NB + b
            k = j + LEAD
            bk = (b + LEAD) % NB

            @pl.when(jnp.logical_and(k >= NB, k < NCHUNK))
            def _():
                scatter(k - NB, bk).wait()
                gather(k, bk).start()

            gather(j, b).wait()
            _scale_chunk(rows.at[b])
            scatter(j, b).start()
        return carry

    lax.fori_loop(0, NCHUNK // NB, outer, 0)

    for b in range(NB):
        scatter(0, b).wait()


@jax.jit
def _embed_scaled(x_w, table):
    k = pl.kernel(
        _sc_body,
        out_type=jax.ShapeDtypeStruct((SEQ, BATCH, D_MODEL), jnp.float32),
        mesh=plsc.VectorSubcoreMesh(core_axis_name="c", subcore_axis_name="s"),
        scratch_types=(
            [pltpu.VMEM((NCHUNK, 2, C // 2), jnp.int32),
             pltpu.VMEM((NB, C, D_MODEL), jnp.float32)]
            + [pltpu.SemaphoreType.DMA] * (3 * NB)
        ),
    )
    return k(x_w, table)


def kernel(x, target_vec, table, W, b):
    bsz, seq = x.shape
    x_w = jnp.transpose(
        x.astype(jnp.int32).T.reshape(seq, NW, C), (1, 0, 2)
    ).reshape(NW, seq, 2, C // 2)
    out_sm = _embed_scaled(x_w, table)
    return jnp.transpose(out_sm, (1, 0, 2))


# speedup vs baseline: 1.0218x; 1.0218x over previous
import math

import jax
import jax.numpy as jnp
from jax import lax
from jax.experimental import pallas as pl
from jax.experimental.pallas import tpu as pltpu
from jax.experimental.pallas import tpu_sc as plsc

D_MODEL = 128
SEQ = 50
SCALE = math.sqrt(128.0)

NC, NS = 2, 16
NW = NC * NS
C = 128
NCHUNK = SEQ
NB = 5
LEAD = 4
BATCH = NW * C


def _scale_chunk(buf):
    def row(i, carry):
        for h in range(D_MODEL // 16):
            sl = (i, pl.ds(h * 16, 16))
            buf[sl] = buf[sl] * SCALE
        return carry
    lax.fori_loop(0, C, row, 0, unroll=2)


def _sc_body(x_hbm, table_hbm, out_hbm, idx_v, rows, *sems):
    gs = sems[:NB]
    ss = sems[NB:]
    wid = lax.axis_index("s") * NC + lax.axis_index("c")
    base = wid * C

    pltpu.sync_copy(x_hbm.at[wid], idx_v)

    def gather(j, b):
        return pltpu.make_async_copy(
            table_hbm.at[idx_v.at[j]], rows.at[b], gs[b])

    def scatter(j, b):
        return pltpu.make_async_copy(
            rows.at[b], out_hbm.at[j, pl.ds(base, C)], ss[b])

    for b in range(NB):
        gather(b, b).start()

    def outer(g, carry):
        for b in range(NB):
            j = g * NB + b
            k = j + LEAD
            bk = (b + LEAD) % NB

            @pl.when(jnp.logical_and(k >= NB, k < NCHUNK))
            def _():
                scatter(k - NB, bk).wait()
                gather(k, bk).start()

            gather(j, b).wait()
            _scale_chunk(rows.at[b])
            scatter(j, b).start()
        return carry

    lax.fori_loop(0, NCHUNK // NB, outer, 0)

    for b in range(NB):
        scatter(0, b).wait()


@jax.jit
def _embed_scaled(x_w, table):
    k = pl.kernel(
        _sc_body,
        out_type=jax.ShapeDtypeStruct((SEQ, BATCH, D_MODEL), jnp.float32),
        mesh=plsc.VectorSubcoreMesh(core_axis_name="c", subcore_axis_name="s"),
        scratch_types=(
            [pltpu.VMEM((NCHUNK, C), jnp.int32),
             pltpu.VMEM((NB, C, D_MODEL), jnp.float32)]
            + [pltpu.SemaphoreType.DMA] * (2 * NB)
        ),
    )
    return k(x_w, table)


def kernel(x, target_vec, table, W, b):
    bsz, seq = x.shape
    x_w = jnp.transpose(
        x.astype(jnp.int32).T.reshape(seq, NW, C), (1, 0, 2))
    out_sm = _embed_scaled(x_w, table)
    return jnp.transpose(out_sm, (1, 0, 2))
